# baseline (speedup 1.0000x reference)
import jax
import jax.numpy as jnp
from jax import lax
from jax.experimental import pallas as pl
from jax.experimental.pallas import tpu as pltpu
from jax.experimental.pallas import tpu_sc as plsc

N = 10000
E = 320000
D = 128
NPAD = 10112
NBLK = NPAD // 128
NTILES = 32
CHUNK = 64
NCHUNK = 160
TPT = NCHUNK * CHUNK
EPAD = NTILES * TPT
SUB = 16
RPT = NPAD // SUB


def _k1_body(x_ref, w_ref, aa_ref, h_ref, at_ref, m_ref, acc_ref):
    i = pl.program_id(0)
    h = jnp.dot(x_ref[...], w_ref[...], preferred_element_type=jnp.float32)
    row = lax.broadcasted_iota(jnp.int32, (128, 128), 0) + i * 128
    h = jnp.where(row < N, h, 0.0)
    h_ref[...] = h
    at = lax.dot_general(aa_ref[...], h, (((0,), (1,)), ((), ())),
                         preferred_element_type=jnp.float32)
    at_ref[...] = at
    bmax = jnp.max(at[0, :]) + jnp.max(at[1, :])

    @pl.when(i == 0)
    def _():
        acc_ref[0] = 0.0

    acc_ref[0] = jnp.maximum(acc_ref[0], bmax)

    @pl.when(i == NBLK - 1)
    def _():
        m_ref[0, 0] = acc_ref[0]


def _k1(x, w, aa):
    return pl.pallas_call(
        _k1_body,
        grid=(NBLK,),
        in_specs=[
            pl.BlockSpec((128, 128), lambda i: (i, 0)),
            pl.BlockSpec((128, 128), lambda i: (0, 0)),
            pl.BlockSpec((128, 8), lambda i: (0, 0)),
        ],
        out_specs=[
            pl.BlockSpec((128, 128), lambda i: (i, 0)),
            pl.BlockSpec((8, 128), lambda i: (0, i)),
            pl.BlockSpec(memory_space=pltpu.SMEM),
        ],
        out_shape=[
            jax.ShapeDtypeStruct((NPAD, 128), jnp.float32),
            jax.ShapeDtypeStruct((8, NPAD), jnp.float32),
            jax.ShapeDtypeStruct((1, 1), jnp.float32),
        ],
        scratch_shapes=[pltpu.SMEM((1,), jnp.float32)],
    )(x, w, aa)


def _sc_body(src_hbm, dst_hbm, h_hbm, at_hbm, m_hbm,
             num_out, den_out,
             as_v, ad_v, den_v, m_v,
             src_c0, src_c1, dst_c0, dst_c1, dst_s0, dst_s1, w_v,
             rows0, rows1, num_sh, isem0, isem1, gsem0, gsem1, ssem):
    c = lax.axis_index("c")
    s = lax.axis_index("s")
    wid = c * SUB + s

    src_c = (src_c0, src_c1)
    dst_c = (dst_c0, dst_c1)
    dst_s = (dst_s0, dst_s1)
    rows = (rows0, rows1)
    isem = (isem0, isem1)
    gsem = (gsem0, gsem1)

    pltpu.sync_copy(at_hbm.at[0], as_v)
    pltpu.sync_copy(at_hbm.at[1], ad_v)
    pltpu.sync_copy(m_hbm, m_v)

    zeros16 = jnp.zeros((16,), jnp.float32)

    @pl.loop(0, NPAD // 16)
    def _(j):
        den_v[pl.ds(j * 16, 16)] = zeros16

    @pl.loop(0, CHUNK)
    def _(r):
        for j in range(8):
            rows1[r, pl.ds(j * 16, 16)] = zeros16

    base = s * RPT

    @pl.loop(0, RPT // CHUNK)
    def _(p):
        pltpu.sync_copy(rows1, num_sh.at[pl.ds(base + p * CHUNK, CHUNK)])

    pltpu.sync_copy(rows1.at[pl.ds(0, RPT - (RPT // CHUNK) * CHUNK)],
                    num_sh.at[pl.ds(base + (RPT // CHUNK) * CHUNK,
                                    RPT - (RPT // CHUNK) * CHUNK)])

    pltpu.sync_copy(src_hbm.at[wid, pl.ds(0, CHUNK)], src_c0)
    pltpu.sync_copy(dst_hbm.at[wid, pl.ds(0, CHUNK)], dst_c0)
    pltpu.async_copy(h_hbm.at[src_c0], rows0, gsem0)
    plsc.subcore_barrier()

    mvec = m_v[...]

    def chunk_step(k, cb, first, last):
        co = 1 - cb
        if not last:
            sl1 = pl.ds((k + 1) * CHUNK, CHUNK)
            pltpu.async_copy(src_hbm.at[wid, sl1], src_c[co], isem[co])
            pltpu.async_copy(dst_hbm.at[wid, sl1], dst_c[co], isem[co])
        for j in range(CHUNK // 16):
            sl = pl.ds(j * 16, 16)
            s16 = src_c[cb][sl]
            d16 = dst_c[cb][sl]
            t = (plsc.load_gather(as_v, [s16]) +
                 plsc.load_gather(ad_v, [d16]))
            e = jnp.where(t >= 0.0, t, t * jnp.float32(0.2))
            w = jnp.exp(e - mvec)
            w_v[sl] = w
            dst_s[cb][sl] = d16
            plsc.addupdate_scatter(den_v, [d16], w)
        if not first:
            pltpu.make_async_copy(rows[co], num_sh.at[dst_s[co]],
                                  ssem).wait()
        pltpu.make_async_copy(h_hbm.at[src_c[cb]], rows[cb],
                              gsem[cb]).wait()
        if not last:
            pltpu.make_async_copy(src_hbm.at[wid, sl1], src_c[co],
                                  isem[co]).wait()
            pltpu.make_async_copy(dst_hbm.at[wid, sl1], dst_c[co],
                                  isem[co]).wait()
            pltpu.async_copy(h_hbm.at[src_c[co]], rows[co], gsem[co])
        @pl.loop(0, CHUNK)
        def _(r):
            wr = plsc.load_gather(w_v, [jnp.full((16,), r, jnp.int32)])
            for j in range(8):
                sl = pl.ds(j * 16, 16)
                rows[cb][r, sl] = rows[cb][r, sl] * wr

        pltpu.async_copy(rows[cb], num_sh.at[dst_s[cb]], ssem, add=True)

    chunk_step(0, 0, first=True, last=False)

    @pl.loop(0, (NCHUNK - 2) // 2)
    def _(i):
        k = i * 2 + 1
        chunk_step(k, 1, first=False, last=False)
        chunk_step(k + 1, 0, first=False, last=False)

    chunk_step(NCHUNK - 1, 1, first=False, last=True)
    pltpu.make_async_copy(rows1, num_sh.at[dst_s1], ssem).wait()
    plsc.subcore_barrier()

    pltpu.sync_copy(den_v, den_out.at[wid])

    @pl.loop(0, RPT // CHUNK)
    def _(p):
        pltpu.sync_copy(num_sh.at[pl.ds(base + p * CHUNK, CHUNK)],
                        num_out.at[c, pl.ds(base + p * CHUNK, CHUNK)])

    tail = RPT - (RPT // CHUNK) * CHUNK
    pltpu.sync_copy(num_sh.at[pl.ds(base + (RPT // CHUNK) * CHUNK, tail)],
                    num_out.at[c, pl.ds(base + (RPT // CHUNK) * CHUNK,
                                        tail)])


def _sc_edge(src2, dst2, h, at, mvec):
    mesh = plsc.VectorSubcoreMesh(core_axis_name="c", subcore_axis_name="s")
    fn = pl.kernel(
        _sc_body,
        out_type=[
            jax.ShapeDtypeStruct((2, NPAD, 128), jnp.float32),
            jax.ShapeDtypeStruct((NTILES, NPAD), jnp.float32),
        ],
        mesh=mesh,
        compiler_params=pltpu.CompilerParams(needs_layout_passes=False),
        scratch_types=[
            pltpu.VMEM((NPAD,), jnp.float32),
            pltpu.VMEM((NPAD,), jnp.float32),
            pltpu.VMEM((NPAD,), jnp.float32),
            pltpu.VMEM((16,), jnp.float32),
            pltpu.VMEM((CHUNK,), jnp.int32),
            pltpu.VMEM((CHUNK,), jnp.int32),
            pltpu.VMEM((CHUNK,), jnp.int32),
            pltpu.VMEM((CHUNK,), jnp.int32),
            pltpu.VMEM((CHUNK,), jnp.int32),
            pltpu.VMEM((CHUNK,), jnp.int32),
            pltpu.VMEM((CHUNK,), jnp.float32),
            pltpu.VMEM((CHUNK, 128), jnp.float32),
            pltpu.VMEM((CHUNK, 128), jnp.float32),
            pltpu.VMEM_SHARED((NPAD, 128), jnp.float32),
            pltpu.SemaphoreType.DMA,
            pltpu.SemaphoreType.DMA,
            pltpu.SemaphoreType.DMA,
            pltpu.SemaphoreType.DMA,
            pltpu.SemaphoreType.DMA,
        ],
    )
    return fn(src2, dst2, h, at, mvec)


def _k2_body(num_ref, den_ref, b_ref, out_ref):
    ssum = num_ref[0] + num_ref[1]
    d = jnp.sum(den_ref[...], axis=0, keepdims=True)
    dcol = jnp.transpose(jnp.broadcast_to(d, (128, 128)))
    out_ref[...] = ssum / (dcol + 1e-16) + b_ref[...]


def _k2(num, den, b):
    return pl.pallas_call(
        _k2_body,
        grid=(NBLK,),
        in_specs=[
            pl.BlockSpec((2, 128, 128), lambda i: (0, i, 0)),
            pl.BlockSpec((NTILES, 128), lambda i: (0, i)),
            pl.BlockSpec((1, 128), lambda i: (0, 0)),
        ],
        out_specs=pl.BlockSpec((128, 128), lambda i: (i, 0)),
        out_shape=jax.ShapeDtypeStruct((NPAD, 128), jnp.float32),
    )(num, den, b)


def _layer(x, src2, dst2, w, a_src, a_dst, b):
    aa = jnp.zeros((D, 8), jnp.float32)
    aa = aa.at[:, 0].set(a_src).at[:, 1].set(a_dst)
    h, at, m = _k1(x, w, aa)
    mvec = jnp.broadcast_to(m[0, 0], (16,))
    num, den = _sc_edge(src2, dst2, h, at, mvec)
    return _k2(num, den, b.reshape(1, D))


@jax.jit
def kernel(x, edge_index, W1, a1_src, a1_dst, b1, W2, a2_src, a2_dst, b2):
    ei = edge_index.astype(jnp.int32)
    ei = jnp.pad(ei, ((0, 0), (0, EPAD - E)), constant_values=N)
    src2 = ei[0].reshape(NTILES, TPT)
    dst2 = ei[1].reshape(NTILES, TPT)
    h1 = _layer(x, src2, dst2, W1, a1_src, a1_dst, b1)
    h2 = _layer(h1, src2, dst2, W2, a2_src, a2_dst, b2)
    return h2[:N]


# speedup vs baseline: 1.0037x; 1.0037x over previous
import jax
import jax.numpy as jnp
from jax import lax
from jax.experimental import pallas as pl
from jax.experimental.pallas import tpu as pltpu
from jax.experimental.pallas import tpu_sc as plsc

N = 10000
E = 320000
D = 128
NPAD = 10112
NBLK = NPAD // 128
NTILES = 32
CHUNK = 64
NCHUNK = 160
TPT = NCHUNK * CHUNK
EPAD = NTILES * TPT
SUB = 16
RPT = NPAD // SUB


def _k1_body(x_ref, w_ref, aa_ref, h_ref, at_ref, m_ref, acc_ref):
    i = pl.program_id(0)
    h = jnp.dot(x_ref[...], w_ref[...], preferred_element_type=jnp.float32)
    row = lax.broadcasted_iota(jnp.int32, (128, 128), 0) + i * 128
    h = jnp.where(row < N, h, 0.0)
    h_ref[...] = h
    at = lax.dot_general(aa_ref[...], h, (((0,), (1,)), ((), ())),
                         preferred_element_type=jnp.float32)
    at_ref[...] = at
    bmax = jnp.max(at[0, :]) + jnp.max(at[1, :])

    @pl.when(i == 0)
    def _():
        acc_ref[0] = 0.0

    acc_ref[0] = jnp.maximum(acc_ref[0], bmax)

    @pl.when(i == NBLK - 1)
    def _():
        m_ref[0, 0] = acc_ref[0]


def _k1(x, w, aa):
    return pl.pallas_call(
        _k1_body,
        grid=(NBLK,),
        in_specs=[
            pl.BlockSpec((128, 128), lambda i: (i, 0)),
            pl.BlockSpec((128, 128), lambda i: (0, 0)),
            pl.BlockSpec((128, 8), lambda i: (0, 0)),
        ],
        out_specs=[
            pl.BlockSpec((128, 128), lambda i: (i, 0)),
            pl.BlockSpec((8, 128), lambda i: (0, i)),
            pl.BlockSpec(memory_space=pltpu.SMEM),
        ],
        out_shape=[
            jax.ShapeDtypeStruct((NPAD, 128), jnp.float32),
            jax.ShapeDtypeStruct((8, NPAD), jnp.float32),
            jax.ShapeDtypeStruct((1, 1), jnp.float32),
        ],
        scratch_shapes=[pltpu.SMEM((1,), jnp.float32)],
    )(x, w, aa)


def _sc_body(src_hbm, dst_hbm, h_hbm, at_hbm, m_hbm,
             num_out, den_out,
             as_v, ad_v, den_v, m_v,
             src_c0, src_c1, dst_c0, dst_c1, dst_s0, dst_s1, w_v,
             rows0, rows1, num_sh, isem0, isem1, gsem0, gsem1, ssem):
    c = lax.axis_index("c")
    s = lax.axis_index("s")
    wid = c * SUB + s

    src_c = (src_c0, src_c1)
    dst_c = (dst_c0, dst_c1)
    dst_s = (dst_s0, dst_s1)
    rows = (rows0, rows1)
    isem = (isem0, isem1)
    gsem = (gsem0, gsem1)

    pltpu.sync_copy(at_hbm.at[0], as_v)
    pltpu.sync_copy(at_hbm.at[1], ad_v)
    pltpu.sync_copy(m_hbm, m_v)

    zeros16 = jnp.zeros((16,), jnp.float32)

    @pl.loop(0, NPAD // 16)
    def _(j):
        den_v[pl.ds(j * 16, 16)] = zeros16

    @pl.loop(0, CHUNK)
    def _(r):
        for j in range(8):
            rows1[r, pl.ds(j * 16, 16)] = zeros16

    base = s * RPT

    @pl.loop(0, RPT // CHUNK)
    def _(p):
        pltpu.sync_copy(rows1, num_sh.at[pl.ds(base + p * CHUNK, CHUNK)])

    pltpu.sync_copy(rows1.at[pl.ds(0, RPT - (RPT // CHUNK) * CHUNK)],
                    num_sh.at[pl.ds(base + (RPT // CHUNK) * CHUNK,
                                    RPT - (RPT // CHUNK) * CHUNK)])

    pltpu.sync_copy(src_hbm.at[wid, pl.ds(0, CHUNK)], src_c0)
    pltpu.sync_copy(dst_hbm.at[wid, pl.ds(0, CHUNK)], dst_c0)
    pltpu.async_copy(h_hbm.at[src_c0], rows0, gsem0)
    plsc.subcore_barrier()

    mvec = m_v[...]

    def chunk_step(k, cb, first, last):
        co = 1 - cb
        if not last:
            sl1 = pl.ds((k + 1) * CHUNK, CHUNK)
            pltpu.async_copy(src_hbm.at[wid, sl1], src_c[co], isem[co])
            pltpu.async_copy(dst_hbm.at[wid, sl1], dst_c[co], isem[co])
        for j in range(CHUNK // 16):
            sl = pl.ds(j * 16, 16)
            s16 = src_c[cb][sl]
            d16 = dst_c[cb][sl]
            t = (plsc.load_gather(as_v, [s16]) +
                 plsc.load_gather(ad_v, [d16]))
            e = jnp.where(t >= 0.0, t, t * jnp.float32(0.2))
            w = jnp.exp(e - mvec)
            w_v[sl] = w
            dst_s[cb][sl] = d16
            plsc.addupdate_scatter(den_v, [d16], w)
        if not first:
            pltpu.make_async_copy(rows[co], num_sh.at[dst_s[co]],
                                  ssem).wait()
        pltpu.make_async_copy(h_hbm.at[src_c[cb]], rows[cb],
                              gsem[cb]).wait()
        if not last:
            pltpu.make_async_copy(src_hbm.at[wid, sl1], src_c[co],
                                  isem[co]).wait()
            pltpu.make_async_copy(dst_hbm.at[wid, sl1], dst_c[co],
                                  isem[co]).wait()
            pltpu.async_copy(h_hbm.at[src_c[co]], rows[co], gsem[co])
        @pl.loop(0, CHUNK // 16)
        def _(g):
            wvec = w_v[pl.ds(g * 16, 16)]
            for r in range(16):
                wr = wvec.at[jnp.full((16,), r, jnp.int32)].get(
                    mode="promise_in_bounds")
                row = g * 16 + r
                for j in range(8):
                    sl = pl.ds(j * 16, 16)
                    rows[cb][row, sl] = rows[cb][row, sl] * wr

        pltpu.async_copy(rows[cb], num_sh.at[dst_s[cb]], ssem, add=True)

    chunk_step(0, 0, first=True, last=False)

    @pl.loop(0, (NCHUNK - 2) // 2)
    def _(i):
        k = i * 2 + 1
        chunk_step(k, 1, first=False, last=False)
        chunk_step(k + 1, 0, first=False, last=False)

    chunk_step(NCHUNK - 1, 1, first=False, last=True)
    pltpu.make_async_copy(rows1, num_sh.at[dst_s1], ssem).wait()
    plsc.subcore_barrier()

    pltpu.sync_copy(den_v, den_out.at[wid])

    @pl.loop(0, RPT // CHUNK)
    def _(p):
        pltpu.sync_copy(num_sh.at[pl.ds(base + p * CHUNK, CHUNK)],
                        num_out.at[c, pl.ds(base + p * CHUNK, CHUNK)])

    tail = RPT - (RPT // CHUNK) * CHUNK
    pltpu.sync_copy(num_sh.at[pl.ds(base + (RPT // CHUNK) * CHUNK, tail)],
                    num_out.at[c, pl.ds(base + (RPT // CHUNK) * CHUNK,
                                        tail)])


def _sc_edge(src2, dst2, h, at, mvec):
    mesh = plsc.VectorSubcoreMesh(core_axis_name="c", subcore_axis_name="s")
    fn = pl.kernel(
        _sc_body,
        out_type=[
            jax.ShapeDtypeStruct((2, NPAD, 128), jnp.float32),
            jax.ShapeDtypeStruct((NTILES, NPAD), jnp.float32),
        ],
        mesh=mesh,
        compiler_params=pltpu.CompilerParams(needs_layout_passes=False),
        scratch_types=[
            pltpu.VMEM((NPAD,), jnp.float32),
            pltpu.VMEM((NPAD,), jnp.float32),
            pltpu.VMEM((NPAD,), jnp.float32),
            pltpu.VMEM((16,), jnp.float32),
            pltpu.VMEM((CHUNK,), jnp.int32),
            pltpu.VMEM((CHUNK,), jnp.int32),
            pltpu.VMEM((CHUNK,), jnp.int32),
            pltpu.VMEM((CHUNK,), jnp.int32),
            pltpu.VMEM((CHUNK,), jnp.int32),
            pltpu.VMEM((CHUNK,), jnp.int32),
            pltpu.VMEM((CHUNK,), jnp.float32),
            pltpu.VMEM((CHUNK, 128), jnp.float32),
            pltpu.VMEM((CHUNK, 128), jnp.float32),
            pltpu.VMEM_SHARED((NPAD, 128), jnp.float32),
            pltpu.SemaphoreType.DMA,
            pltpu.SemaphoreType.DMA,
            pltpu.SemaphoreType.DMA,
            pltpu.SemaphoreType.DMA,
            pltpu.SemaphoreType.DMA,
        ],
    )
    return fn(src2, dst2, h, at, mvec)


def _k2_body(num_ref, den_ref, b_ref, out_ref):
    ssum = num_ref[0] + num_ref[1]
    d = jnp.sum(den_ref[...], axis=0, keepdims=True)
    dcol = jnp.transpose(jnp.broadcast_to(d, (128, 128)))
    out_ref[...] = ssum / (dcol + 1e-16) + b_ref[...]


def _k2(num, den, b):
    return pl.pallas_call(
        _k2_body,
        grid=(NBLK,),
        in_specs=[
            pl.BlockSpec((2, 128, 128), lambda i: (0, i, 0)),
            pl.BlockSpec((NTILES, 128), lambda i: (0, i)),
            pl.BlockSpec((1, 128), lambda i: (0, 0)),
        ],
        out_specs=pl.BlockSpec((128, 128), lambda i: (i, 0)),
        out_shape=jax.ShapeDtypeStruct((NPAD, 128), jnp.float32),
    )(num, den, b)


def _layer(x, src2, dst2, w, a_src, a_dst, b):
    aa = jnp.zeros((D, 8), jnp.float32)
    aa = aa.at[:, 0].set(a_src).at[:, 1].set(a_dst)
    h, at, m = _k1(x, w, aa)
    mvec = jnp.broadcast_to(m[0, 0], (16,))
    num, den = _sc_edge(src2, dst2, h, at, mvec)
    return _k2(num, den, b.reshape(1, D))


@jax.jit
def kernel(x, edge_index, W1, a1_src, a1_dst, b1, W2, a2_src, a2_dst, b2):
    ei = edge_index.astype(jnp.int32)
    ei = jnp.pad(ei, ((0, 0), (0, EPAD - E)), constant_values=N)
    src2 = ei[0].reshape(NTILES, TPT)
    dst2 = ei[1].reshape(NTILES, TPT)
    h1 = _layer(x, src2, dst2, W1, a1_src, a1_dst, b1)
    h2 = _layer(h1, src2, dst2, W2, a2_src, a2_dst, b2)
    return h2[:N]


# speedup vs baseline: 1.5039x; 1.4983x over previous
import jax
import jax.numpy as jnp
from jax import lax
from jax.experimental import pallas as pl
from jax.experimental.pallas import tpu as pltpu
from jax.experimental.pallas import tpu_sc as plsc

N = 10000
E = 320000
D = 128
NPAD = 10112
NBLK = NPAD // 128
NTILES = 32
CHUNK = 32
NCHUNK = 316
TPT = NCHUNK * CHUNK
EPAD = NTILES * TPT
SUB = 16
RPT = NPAD // SUB


def _k1_body(x_ref, w_ref, aa_ref, h_ref, at_ref, m_ref, acc_ref):
    i = pl.program_id(0)
    h = jnp.dot(x_ref[...], w_ref[...], preferred_element_type=jnp.float32)
    row = lax.broadcasted_iota(jnp.int32, (128, 128), 0) + i * 128
    h = jnp.where(row < N, h, 0.0)
    h_ref[...] = h
    at = lax.dot_general(aa_ref[...], h, (((0,), (1,)), ((), ())),
                         preferred_element_type=jnp.float32)
    at_ref[...] = at
    bmax = jnp.max(at[0, :]) + jnp.max(at[1, :])

    @pl.when(i == 0)
    def _():
        acc_ref[0] = 0.0

    acc_ref[0] = jnp.maximum(acc_ref[0], bmax)

    @pl.when(i == NBLK - 1)
    def _():
        m_ref[0, 0] = acc_ref[0]


def _k1(x, w, aa):
    return pl.pallas_call(
        _k1_body,
        grid=(NBLK,),
        in_specs=[
            pl.BlockSpec((128, 128), lambda i: (i, 0)),
            pl.BlockSpec((128, 128), lambda i: (0, 0)),
            pl.BlockSpec((128, 8), lambda i: (0, 0)),
        ],
        out_specs=[
            pl.BlockSpec((128, 128), lambda i: (i, 0)),
            pl.BlockSpec((8, 128), lambda i: (0, i)),
            pl.BlockSpec(memory_space=pltpu.SMEM),
        ],
        out_shape=[
            jax.ShapeDtypeStruct((NPAD, 128), jnp.float32),
            jax.ShapeDtypeStruct((8, NPAD), jnp.float32),
            jax.ShapeDtypeStruct((1, 1), jnp.float32),
        ],
        scratch_shapes=[pltpu.SMEM((1,), jnp.float32)],
    )(x, w, aa)


def _sc_body(src_hbm, dst_hbm, h_hbm, at_hbm, m_hbm,
             num_out, den_out,
             as_v, ad_v, den_v, m_v,
             src_c0, src_c1, src_c2, src_c3,
             dst_c0, dst_c1, dst_c2, dst_c3,
             dst_s0, dst_s1, dst_s2, dst_s3, w_v,
             rows0, rows1, rows2, rows3, num_sh,
             isem0, isem1, isem2, isem3,
             gsem0, gsem1, gsem2, gsem3,
             ssem0, ssem1, ssem2, ssem3):
    c = lax.axis_index("c")
    s = lax.axis_index("s")
    wid = c * SUB + s

    src_c = (src_c0, src_c1, src_c2, src_c3)
    dst_c = (dst_c0, dst_c1, dst_c2, dst_c3)
    dst_s = (dst_s0, dst_s1, dst_s2, dst_s3)
    rows = (rows0, rows1, rows2, rows3)
    isem = (isem0, isem1, isem2, isem3)
    gsem = (gsem0, gsem1, gsem2, gsem3)
    ssem = (ssem0, ssem1, ssem2, ssem3)

    pltpu.sync_copy(at_hbm.at[0], as_v)
    pltpu.sync_copy(at_hbm.at[1], ad_v)
    pltpu.sync_copy(m_hbm, m_v)

    zeros16 = jnp.zeros((16,), jnp.float32)

    @pl.loop(0, NPAD // 16)
    def _(j):
        den_v[pl.ds(j * 16, 16)] = zeros16

    @pl.loop(0, CHUNK)
    def _(r):
        for j in range(8):
            rows3[r, pl.ds(j * 16, 16)] = zeros16

    base = s * RPT

    @pl.loop(0, RPT // CHUNK)
    def _(p):
        pltpu.sync_copy(rows3, num_sh.at[pl.ds(base + p * CHUNK, CHUNK)])

    tail0 = RPT - (RPT // CHUNK) * CHUNK
    pltpu.sync_copy(rows3.at[pl.ds(0, tail0)],
                    num_sh.at[pl.ds(base + (RPT // CHUNK) * CHUNK, tail0)])

    pltpu.sync_copy(src_hbm.at[wid, pl.ds(0, CHUNK)], src_c0)
    pltpu.sync_copy(dst_hbm.at[wid, pl.ds(0, CHUNK)], dst_c0)
    pltpu.sync_copy(src_hbm.at[wid, pl.ds(CHUNK, CHUNK)], src_c1)
    pltpu.sync_copy(dst_hbm.at[wid, pl.ds(CHUNK, CHUNK)], dst_c1)
    for kk in (2, 3):
        slk = pl.ds(kk * CHUNK, CHUNK)
        pltpu.async_copy(src_hbm.at[wid, slk], src_c[kk], isem[kk])
        pltpu.async_copy(dst_hbm.at[wid, slk], dst_c[kk], isem[kk])
    pltpu.async_copy(h_hbm.at[src_c0], rows0, gsem0)
    pltpu.async_copy(h_hbm.at[src_c1], rows1, gsem1)
    plsc.subcore_barrier()

    mvec = m_v[...]

    def chunk_step(k, b, drain, load4, issue2):
        bd = (b + 2) % 4
        for j in range(CHUNK // 16):
            sl = pl.ds(j * 16, 16)
            s16 = src_c[b][sl]
            d16 = dst_c[b][sl]
            t = (plsc.load_gather(as_v, [s16]) +
                 plsc.load_gather(ad_v, [d16]))
            e = jnp.where(t >= 0.0, t, t * jnp.float32(0.2))
            w = jnp.exp(e - mvec)
            w_v[sl] = w
            dst_s[b][sl] = d16
            plsc.addupdate_scatter(den_v, [d16], w)
        if drain:
            pltpu.make_async_copy(rows[bd], num_sh.at[dst_s[bd]],
                                  ssem[bd]).wait()
        pltpu.make_async_copy(h_hbm.at[src_c[b]], rows[b], gsem[b]).wait()
        if load4:
            sl4 = pl.ds((k + 4) * CHUNK, CHUNK)
            pltpu.async_copy(src_hbm.at[wid, sl4], src_c[b], isem[b])
            pltpu.async_copy(dst_hbm.at[wid, sl4], dst_c[b], isem[b])
        if issue2:
            sl2 = pl.ds((k + 2) * CHUNK, CHUNK)
            pltpu.make_async_copy(src_hbm.at[wid, sl2], src_c[bd],
                                  isem[bd]).wait()
            pltpu.make_async_copy(dst_hbm.at[wid, sl2], dst_c[bd],
                                  isem[bd]).wait()
            pltpu.async_copy(h_hbm.at[src_c[bd]], rows[bd], gsem[bd])
        @pl.loop(0, CHUNK // 16)
        def _(g):
            wvec = w_v[pl.ds(g * 16, 16)]
            for r in range(16):
                wr = wvec.at[jnp.full((16,), r, jnp.int32)].get(
                    mode="promise_in_bounds")
                row = g * 16 + r
                for j in range(8):
                    sl = pl.ds(j * 16, 16)
                    rows[b][row, sl] = rows[b][row, sl] * wr

        pltpu.async_copy(rows[b], num_sh.at[dst_s[b]], ssem[b], add=True)

    chunk_step(0, 0, drain=False, load4=True, issue2=True)
    chunk_step(1, 1, drain=False, load4=True, issue2=True)
    chunk_step(2, 2, drain=True, load4=True, issue2=True)
    chunk_step(3, 3, drain=True, load4=True, issue2=True)

    @pl.loop(0, (NCHUNK - 8) // 4)
    def _(i):
        k = i * 4 + 4
        chunk_step(k, 0, drain=True, load4=True, issue2=True)
        chunk_step(k + 1, 1, drain=True, load4=True, issue2=True)
        chunk_step(k + 2, 2, drain=True, load4=True, issue2=True)
        chunk_step(k + 3, 3, drain=True, load4=True, issue2=True)

    chunk_step(NCHUNK - 4, 0, drain=True, load4=False, issue2=True)
    chunk_step(NCHUNK - 3, 1, drain=True, load4=False, issue2=True)
    chunk_step(NCHUNK - 2, 2, drain=True, load4=False, issue2=False)
    chunk_step(NCHUNK - 1, 3, drain=True, load4=False, issue2=False)
    pltpu.make_async_copy(rows2, num_sh.at[dst_s2], ssem2).wait()
    pltpu.make_async_copy(rows3, num_sh.at[dst_s3], ssem3).wait()
    plsc.subcore_barrier()

    pltpu.sync_copy(den_v, den_out.at[wid])

    @pl.loop(0, RPT // CHUNK)
    def _(p):
        pltpu.sync_copy(num_sh.at[pl.ds(base + p * CHUNK, CHUNK)],
                        num_out.at[c, pl.ds(base + p * CHUNK, CHUNK)])

    pltpu.sync_copy(num_sh.at[pl.ds(base + (RPT // CHUNK) * CHUNK, tail0)],
                    num_out.at[c, pl.ds(base + (RPT // CHUNK) * CHUNK,
                                        tail0)])


def _sc_edge(src2, dst2, h, at, mvec):
    mesh = plsc.VectorSubcoreMesh(core_axis_name="c", subcore_axis_name="s")
    fn = pl.kernel(
        _sc_body,
        out_type=[
            jax.ShapeDtypeStruct((2, NPAD, 128), jnp.float32),
            jax.ShapeDtypeStruct((NTILES, NPAD), jnp.float32),
        ],
        mesh=mesh,
        compiler_params=pltpu.CompilerParams(needs_layout_passes=False),
        scratch_types=[
            pltpu.VMEM((NPAD,), jnp.float32),
            pltpu.VMEM((NPAD,), jnp.float32),
            pltpu.VMEM((NPAD,), jnp.float32),
            pltpu.VMEM((16,), jnp.float32),
        ] + [pltpu.VMEM((CHUNK,), jnp.int32) for _ in range(12)] + [
            pltpu.VMEM((CHUNK,), jnp.float32),
            pltpu.VMEM((CHUNK, 128), jnp.float32),
            pltpu.VMEM((CHUNK, 128), jnp.float32),
            pltpu.VMEM((CHUNK, 128), jnp.float32),
            pltpu.VMEM((CHUNK, 128), jnp.float32),
            pltpu.VMEM_SHARED((NPAD, 128), jnp.float32),
        ] + [pltpu.SemaphoreType.DMA for _ in range(12)],
    )
    return fn(src2, dst2, h, at, mvec)


def _k2_body(num_ref, den_ref, b_ref, out_ref):
    ssum = num_ref[0] + num_ref[1]
    d = jnp.sum(den_ref[...], axis=0, keepdims=True)
    dcol = jnp.transpose(jnp.broadcast_to(d, (128, 128)))
    out_ref[...] = ssum / (dcol + 1e-16) + b_ref[...]


def _k2(num, den, b):
    return pl.pallas_call(
        _k2_body,
        grid=(NBLK,),
        in_specs=[
            pl.BlockSpec((2, 128, 128), lambda i: (0, i, 0)),
            pl.BlockSpec((NTILES, 128), lambda i: (0, i)),
            pl.BlockSpec((1, 128), lambda i: (0, 0)),
        ],
        out_specs=pl.BlockSpec((128, 128), lambda i: (i, 0)),
        out_shape=jax.ShapeDtypeStruct((NPAD, 128), jnp.float32),
    )(num, den, b)


def _layer(x, src2, dst2, w, a_src, a_dst, b):
    aa = jnp.zeros((D, 8), jnp.float32)
    aa = aa.at[:, 0].set(a_src).at[:, 1].set(a_dst)
    h, at, m = _k1(x, w, aa)
    mvec = jnp.broadcast_to(m[0, 0], (16,))
    num, den = _sc_edge(src2, dst2, h, at, mvec)
    return _k2(num, den, b.reshape(1, D))


@jax.jit
def kernel(x, edge_index, W1, a1_src, a1_dst, b1, W2, a2_src, a2_dst, b2):
    ei = edge_index.astype(jnp.int32)
    ei = jnp.pad(ei, ((0, 0), (0, EPAD - E)), constant_values=N)
    src2 = ei[0].reshape(NTILES, TPT)
    dst2 = ei[1].reshape(NTILES, TPT)
    h1 = _layer(x, src2, dst2, W1, a1_src, a1_dst, b1)
    h2 = _layer(h1, src2, dst2, W2, a2_src, a2_dst, b2)
    return h2[:N]


# speedup vs baseline: 1.6445x; 1.0935x over previous
import jax
import jax.numpy as jnp
from jax import lax
from jax.experimental import pallas as pl
from jax.experimental.pallas import tpu as pltpu
from jax.experimental.pallas import tpu_sc as plsc

N = 10000
E = 320000
D = 128
NPAD = 10112
NBLK = NPAD // 128
NTILES = 32
CHUNK = 32
NCHUNK = 316
TPT = NCHUNK * CHUNK
EPAD = NTILES * TPT
SUB = 16
RPT = NPAD // SUB


def _k1_body(x_ref, w_ref, aa_ref, h_ref, at_ref, m_ref, acc_ref):
    acc_ref[0] = 0.0
    wmat = w_ref[...]
    aam = aa_ref[...]

    @pl.loop(0, NBLK)
    def _(i):
        sl = pl.ds(i * 128, 128)
        h = jnp.dot(x_ref[sl, :], wmat, preferred_element_type=jnp.float32)
        row = lax.broadcasted_iota(jnp.int32, (128, 128), 0) + i * 128
        h = jnp.where(row < N, h, 0.0)
        h_ref[sl, :] = h
        at = lax.dot_general(aam, h, (((0,), (1,)), ((), ())),
                             preferred_element_type=jnp.float32)
        at_ref[:, sl] = at
        bmax = jnp.max(at[0, :]) + jnp.max(at[1, :])
        acc_ref[0] = jnp.maximum(acc_ref[0], bmax)

    m_ref[0, 0] = acc_ref[0]


def _k1(x, w, aa):
    return pl.pallas_call(
        _k1_body,
        out_shape=[
            jax.ShapeDtypeStruct((NPAD, 128), jnp.float32),
            jax.ShapeDtypeStruct((8, NPAD), jnp.float32),
            jax.ShapeDtypeStruct((1, 1), jnp.float32),
        ],
        out_specs=[
            pl.BlockSpec((NPAD, 128), lambda: (0, 0)),
            pl.BlockSpec((8, NPAD), lambda: (0, 0)),
            pl.BlockSpec(memory_space=pltpu.SMEM),
        ],
        scratch_shapes=[pltpu.SMEM((1,), jnp.float32)],
    )(x, w, aa)


def _kmid_body(num_ref, den_ref, b_ref, w_ref, aa_ref,
               h_ref, at_ref, m_ref, acc_ref):
    acc_ref[0] = 0.0
    wmat = w_ref[...]
    aam = aa_ref[...]
    bvec = b_ref[...]

    @pl.loop(0, NBLK)
    def _(i):
        sl = pl.ds(i * 128, 128)
        ssum = num_ref[0, sl, :] + num_ref[1, sl, :]
        d = jnp.sum(den_ref[:, sl], axis=0, keepdims=True)
        dcol = jnp.transpose(jnp.broadcast_to(d, (128, 128)))
        x2 = ssum / (dcol + 1e-16) + bvec
        h = jnp.dot(x2, wmat, preferred_element_type=jnp.float32)
        row = lax.broadcasted_iota(jnp.int32, (128, 128), 0) + i * 128
        h = jnp.where(row < N, h, 0.0)
        h_ref[sl, :] = h
        at = lax.dot_general(aam, h, (((0,), (1,)), ((), ())),
                             preferred_element_type=jnp.float32)
        at_ref[:, sl] = at
        bmax = jnp.max(at[0, :]) + jnp.max(at[1, :])
        acc_ref[0] = jnp.maximum(acc_ref[0], bmax)

    m_ref[0, 0] = acc_ref[0]


def _kmid(num, den, b, w, aa):
    return pl.pallas_call(
        _kmid_body,
        out_shape=[
            jax.ShapeDtypeStruct((NPAD, 128), jnp.float32),
            jax.ShapeDtypeStruct((8, NPAD), jnp.float32),
            jax.ShapeDtypeStruct((1, 1), jnp.float32),
        ],
        out_specs=[
            pl.BlockSpec((NPAD, 128), lambda: (0, 0)),
            pl.BlockSpec((8, NPAD), lambda: (0, 0)),
            pl.BlockSpec(memory_space=pltpu.SMEM),
        ],
        scratch_shapes=[pltpu.SMEM((1,), jnp.float32)],
    )(num, den, b, w, aa)


def _sc_body(src_hbm, dst_hbm, h_hbm, at_hbm, m_hbm,
             num_out, den_out,
             as_v, ad_v, den_v, m_v,
             src_c0, src_c1, src_c2, src_c3,
             dst_c0, dst_c1, dst_c2, dst_c3,
             dst_s0, dst_s1, dst_s2, dst_s3, w_v,
             rows0, rows1, rows2, rows3, num_sh,
             isem0, isem1, isem2, isem3,
             gsem0, gsem1, gsem2, gsem3,
             ssem0, ssem1, ssem2, ssem3):
    c = lax.axis_index("c")
    s = lax.axis_index("s")
    wid = c * SUB + s

    src_c = (src_c0, src_c1, src_c2, src_c3)
    dst_c = (dst_c0, dst_c1, dst_c2, dst_c3)
    dst_s = (dst_s0, dst_s1, dst_s2, dst_s3)
    rows = (rows0, rows1, rows2, rows3)
    isem = (isem0, isem1, isem2, isem3)
    gsem = (gsem0, gsem1, gsem2, gsem3)
    ssem = (ssem0, ssem1, ssem2, ssem3)

    pltpu.sync_copy(at_hbm.at[0], as_v)
    pltpu.sync_copy(at_hbm.at[1], ad_v)
    pltpu.sync_copy(m_hbm, m_v)

    zeros16 = jnp.zeros((16,), jnp.float32)

    @pl.loop(0, NPAD // 16)
    def _(j):
        den_v[pl.ds(j * 16, 16)] = zeros16

    @pl.loop(0, CHUNK)
    def _(r):
        for j in range(8):
            rows3[r, pl.ds(j * 16, 16)] = zeros16

    base = s * RPT

    @pl.loop(0, RPT // CHUNK)
    def _(p):
        pltpu.sync_copy(rows3, num_sh.at[pl.ds(base + p * CHUNK, CHUNK)])

    tail0 = RPT - (RPT // CHUNK) * CHUNK
    pltpu.sync_copy(rows3.at[pl.ds(0, tail0)],
                    num_sh.at[pl.ds(base + (RPT // CHUNK) * CHUNK, tail0)])

    pltpu.sync_copy(src_hbm.at[wid, pl.ds(0, CHUNK)], src_c0)
    pltpu.sync_copy(dst_hbm.at[wid, pl.ds(0, CHUNK)], dst_c0)
    pltpu.sync_copy(src_hbm.at[wid, pl.ds(CHUNK, CHUNK)], src_c1)
    pltpu.sync_copy(dst_hbm.at[wid, pl.ds(CHUNK, CHUNK)], dst_c1)
    for kk in (2, 3):
        slk = pl.ds(kk * CHUNK, CHUNK)
        pltpu.async_copy(src_hbm.at[wid, slk], src_c[kk], isem[kk])
        pltpu.async_copy(dst_hbm.at[wid, slk], dst_c[kk], isem[kk])
    pltpu.async_copy(h_hbm.at[src_c0], rows0, gsem0)
    pltpu.async_copy(h_hbm.at[src_c1], rows1, gsem1)
    plsc.subcore_barrier()

    mvec = m_v[...]

    def chunk_step(k, b, drain, load4, issue2):
        bd = (b + 2) % 4
        for j in range(CHUNK // 16):
            sl = pl.ds(j * 16, 16)
            s16 = src_c[b][sl]
            d16 = dst_c[b][sl]
            t = (plsc.load_gather(as_v, [s16]) +
                 plsc.load_gather(ad_v, [d16]))
            e = jnp.where(t >= 0.0, t, t * jnp.float32(0.2))
            w = jnp.exp(e - mvec)
            w_v[sl] = w
            dst_s[b][sl] = d16
            plsc.addupdate_scatter(den_v, [d16], w)
        if drain:
            pltpu.make_async_copy(rows[bd], num_sh.at[dst_s[bd]],
                                  ssem[bd]).wait()
        pltpu.make_async_copy(h_hbm.at[src_c[b]], rows[b], gsem[b]).wait()
        if load4:
            sl4 = pl.ds((k + 4) * CHUNK, CHUNK)
            pltpu.async_copy(src_hbm.at[wid, sl4], src_c[b], isem[b])
            pltpu.async_copy(dst_hbm.at[wid, sl4], dst_c[b], isem[b])
        if issue2:
            sl2 = pl.ds((k + 2) * CHUNK, CHUNK)
            pltpu.make_async_copy(src_hbm.at[wid, sl2], src_c[bd],
                                  isem[bd]).wait()
            pltpu.make_async_copy(dst_hbm.at[wid, sl2], dst_c[bd],
                                  isem[bd]).wait()
            pltpu.async_copy(h_hbm.at[src_c[bd]], rows[bd], gsem[bd])
        @pl.loop(0, CHUNK // 16)
        def _(g):
            wvec = w_v[pl.ds(g * 16, 16)]
            for r in range(16):
                wr = wvec.at[jnp.full((16,), r, jnp.int32)].get(
                    mode="promise_in_bounds")
                row = g * 16 + r
                for j in range(8):
                    sl = pl.ds(j * 16, 16)
                    rows[b][row, sl] = rows[b][row, sl] * wr

        pltpu.async_copy(rows[b], num_sh.at[dst_s[b]], ssem[b], add=True)

    chunk_step(0, 0, drain=False, load4=True, issue2=True)
    chunk_step(1, 1, drain=False, load4=True, issue2=True)
    chunk_step(2, 2, drain=True, load4=True, issue2=True)
    chunk_step(3, 3, drain=True, load4=True, issue2=True)

    @pl.loop(0, (NCHUNK - 8) // 4)
    def _(i):
        k = i * 4 + 4
        chunk_step(k, 0, drain=True, load4=True, issue2=True)
        chunk_step(k + 1, 1, drain=True, load4=True, issue2=True)
        chunk_step(k + 2, 2, drain=True, load4=True, issue2=True)
        chunk_step(k + 3, 3, drain=True, load4=True, issue2=True)

    chunk_step(NCHUNK - 4, 0, drain=True, load4=False, issue2=True)
    chunk_step(NCHUNK - 3, 1, drain=True, load4=False, issue2=True)
    chunk_step(NCHUNK - 2, 2, drain=True, load4=False, issue2=False)
    chunk_step(NCHUNK - 1, 3, drain=True, load4=False, issue2=False)
    pltpu.make_async_copy(rows2, num_sh.at[dst_s2], ssem2).wait()
    pltpu.make_async_copy(rows3, num_sh.at[dst_s3], ssem3).wait()
    plsc.subcore_barrier()

    pltpu.sync_copy(den_v, den_out.at[wid])

    @pl.loop(0, RPT // CHUNK)
    def _(p):
        pltpu.sync_copy(num_sh.at[pl.ds(base + p * CHUNK, CHUNK)],
                        num_out.at[c, pl.ds(base + p * CHUNK, CHUNK)])

    pltpu.sync_copy(num_sh.at[pl.ds(base + (RPT // CHUNK) * CHUNK, tail0)],
                    num_out.at[c, pl.ds(base + (RPT // CHUNK) * CHUNK,
                                        tail0)])


def _sc_edge(src2, dst2, h, at, mvec):
    mesh = plsc.VectorSubcoreMesh(core_axis_name="c", subcore_axis_name="s")
    fn = pl.kernel(
        _sc_body,
        out_type=[
            jax.ShapeDtypeStruct((2, NPAD, 128), jnp.float32),
            jax.ShapeDtypeStruct((NTILES, NPAD), jnp.float32),
        ],
        mesh=mesh,
        compiler_params=pltpu.CompilerParams(needs_layout_passes=False),
        scratch_types=[
            pltpu.VMEM((NPAD,), jnp.float32),
            pltpu.VMEM((NPAD,), jnp.float32),
            pltpu.VMEM((NPAD,), jnp.float32),
            pltpu.VMEM((16,), jnp.float32),
        ] + [pltpu.VMEM((CHUNK,), jnp.int32) for _ in range(12)] + [
            pltpu.VMEM((CHUNK,), jnp.float32),
            pltpu.VMEM((CHUNK, 128), jnp.float32),
            pltpu.VMEM((CHUNK, 128), jnp.float32),
            pltpu.VMEM((CHUNK, 128), jnp.float32),
            pltpu.VMEM((CHUNK, 128), jnp.float32),
            pltpu.VMEM_SHARED((NPAD, 128), jnp.float32),
        ] + [pltpu.SemaphoreType.DMA for _ in range(12)],
    )
    return fn(src2, dst2, h, at, mvec)


def _k2_body(num_ref, den_ref, b_ref, out_ref):
    bvec = b_ref[...]

    @pl.loop(0, NBLK)
    def _(i):
        sl = pl.ds(i * 128, 128)
        ssum = num_ref[0, sl, :] + num_ref[1, sl, :]
        d = jnp.sum(den_ref[:, sl], axis=0, keepdims=True)
        dcol = jnp.transpose(jnp.broadcast_to(d, (128, 128)))
        out_ref[sl, :] = ssum / (dcol + 1e-16) + bvec


def _k2(num, den, b):
    return pl.pallas_call(
        _k2_body,
        out_shape=jax.ShapeDtypeStruct((NPAD, 128), jnp.float32),
        out_specs=pl.BlockSpec((NPAD, 128), lambda: (0, 0)),
    )(num, den, b)


def _pack_aa(a_src, a_dst):
    aa = jnp.zeros((D, 8), jnp.float32)
    return aa.at[:, 0].set(a_src).at[:, 1].set(a_dst)


@jax.jit
def kernel(x, edge_index, W1, a1_src, a1_dst, b1, W2, a2_src, a2_dst, b2):
    ei = edge_index.astype(jnp.int32)
    ei = jnp.pad(ei, ((0, 0), (0, EPAD - E)), constant_values=N)
    src2 = ei[0].reshape(NTILES, TPT)
    dst2 = ei[1].reshape(NTILES, TPT)
    xpad = jnp.pad(x, ((0, NPAD - N), (0, 0)))
    h1, at1, m1 = _k1(xpad, W1, _pack_aa(a1_src, a1_dst))
    num1, den1 = _sc_edge(src2, dst2, h1, at1,
                          jnp.broadcast_to(m1[0, 0], (16,)))
    h2, at2, m2 = _kmid(num1, den1, b1.reshape(1, D), W2,
                        _pack_aa(a2_src, a2_dst))
    num2, den2 = _sc_edge(src2, dst2, h2, at2,
                          jnp.broadcast_to(m2[0, 0], (16,)))
    return _k2(num2, den2, b2.reshape(1, D))[:N]


# speedup vs baseline: 1.7410x; 1.0587x over previous
import jax
import jax.numpy as jnp
from jax import lax
from jax.experimental import pallas as pl
from jax.experimental.pallas import tpu as pltpu
from jax.experimental.pallas import tpu_sc as plsc

N = 10000
E = 320000
D = 128
NPAD = 10112
NBLK = NPAD // 128
NTILES = 32
CHUNK = 16
NCHUNK = 630
TPT = NCHUNK * CHUNK
EPAD = NTILES * TPT
SUB = 16
RPT = NPAD // SUB


def _k1_body(x_ref, w_ref, aa_ref, h_ref, at_ref, m_ref, acc_ref):
    acc_ref[0] = 0.0
    wmat = w_ref[...]
    aam = aa_ref[...]

    @pl.loop(0, NBLK)
    def _(i):
        sl = pl.ds(i * 128, 128)
        h = jnp.dot(x_ref[sl, :], wmat, preferred_element_type=jnp.float32)
        row = lax.broadcasted_iota(jnp.int32, (128, 128), 0) + i * 128
        h = jnp.where(row < N, h, 0.0)
        h_ref[sl, :] = h
        at = lax.dot_general(aam, h, (((0,), (1,)), ((), ())),
                             preferred_element_type=jnp.float32)
        at_ref[:, sl] = at
        bmax = jnp.max(at[0, :]) + jnp.max(at[1, :])
        acc_ref[0] = jnp.maximum(acc_ref[0], bmax)

    m_ref[0, 0] = acc_ref[0]


def _k1(x, w, aa):
    return pl.pallas_call(
        _k1_body,
        out_shape=[
            jax.ShapeDtypeStruct((NPAD, 128), jnp.float32),
            jax.ShapeDtypeStruct((8, NPAD), jnp.float32),
            jax.ShapeDtypeStruct((1, 1), jnp.float32),
        ],
        out_specs=[
            pl.BlockSpec((NPAD, 128), lambda: (0, 0)),
            pl.BlockSpec((8, NPAD), lambda: (0, 0)),
            pl.BlockSpec(memory_space=pltpu.SMEM),
        ],
        scratch_shapes=[pltpu.SMEM((1,), jnp.float32)],
    )(x, w, aa)


def _kmid_body(num_ref, den_ref, b_ref, w_ref, aa_ref,
               h_ref, at_ref, m_ref, acc_ref):
    acc_ref[0] = 0.0
    wmat = w_ref[...]
    aam = aa_ref[...]
    bvec = b_ref[...]

    @pl.loop(0, NBLK)
    def _(i):
        sl = pl.ds(i * 128, 128)
        ssum = num_ref[0, sl, :] + num_ref[1, sl, :]
        d = jnp.sum(den_ref[:, sl], axis=0, keepdims=True)
        dcol = jnp.transpose(jnp.broadcast_to(d, (128, 128)))
        x2 = ssum / (dcol + 1e-16) + bvec
        h = jnp.dot(x2, wmat, preferred_element_type=jnp.float32)
        row = lax.broadcasted_iota(jnp.int32, (128, 128), 0) + i * 128
        h = jnp.where(row < N, h, 0.0)
        h_ref[sl, :] = h
        at = lax.dot_general(aam, h, (((0,), (1,)), ((), ())),
                             preferred_element_type=jnp.float32)
        at_ref[:, sl] = at
        bmax = jnp.max(at[0, :]) + jnp.max(at[1, :])
        acc_ref[0] = jnp.maximum(acc_ref[0], bmax)

    m_ref[0, 0] = acc_ref[0]


def _kmid(num, den, b, w, aa):
    return pl.pallas_call(
        _kmid_body,
        out_shape=[
            jax.ShapeDtypeStruct((NPAD, 128), jnp.float32),
            jax.ShapeDtypeStruct((8, NPAD), jnp.float32),
            jax.ShapeDtypeStruct((1, 1), jnp.float32),
        ],
        out_specs=[
            pl.BlockSpec((NPAD, 128), lambda: (0, 0)),
            pl.BlockSpec((8, NPAD), lambda: (0, 0)),
            pl.BlockSpec(memory_space=pltpu.SMEM),
        ],
        scratch_shapes=[pltpu.SMEM((1,), jnp.float32)],
    )(num, den, b, w, aa)


def _sc_body(src_hbm, dst_hbm, h_hbm, at_hbm, m_hbm,
             num_out, den_out,
             as_v, ad_v, den_v, m_v,
             src_c0, src_c1, src_c2, src_c3, src_c4, src_c5,
             dst_c0, dst_c1, dst_c2, dst_c3, dst_c4, dst_c5,
             dst_s0, dst_s1, dst_s2, dst_s3, dst_s4, dst_s5, w_v,
             rows0, rows1, rows2, rows3, rows4, rows5, num_sh,
             isem0, isem1, isem2, isem3, isem4, isem5,
             gsem0, gsem1, gsem2, gsem3, gsem4, gsem5,
             ssem0, ssem1, ssem2, ssem3, ssem4, ssem5):
    c = lax.axis_index("c")
    s = lax.axis_index("s")
    wid = c * SUB + s

    src_c = (src_c0, src_c1, src_c2, src_c3, src_c4, src_c5)
    dst_c = (dst_c0, dst_c1, dst_c2, dst_c3, dst_c4, dst_c5)
    dst_s = (dst_s0, dst_s1, dst_s2, dst_s3, dst_s4, dst_s5)
    rows = (rows0, rows1, rows2, rows3, rows4, rows5)
    isem = (isem0, isem1, isem2, isem3, isem4, isem5)
    gsem = (gsem0, gsem1, gsem2, gsem3, gsem4, gsem5)
    ssem = (ssem0, ssem1, ssem2, ssem3, ssem4, ssem5)

    pltpu.sync_copy(at_hbm.at[0], as_v)
    pltpu.sync_copy(at_hbm.at[1], ad_v)
    pltpu.sync_copy(m_hbm, m_v)

    zeros16 = jnp.zeros((16,), jnp.float32)

    @pl.loop(0, NPAD // 16)
    def _(j):
        den_v[pl.ds(j * 16, 16)] = zeros16

    @pl.loop(0, CHUNK)
    def _(r):
        for j in range(8):
            rows3[r, pl.ds(j * 16, 16)] = zeros16

    base = s * RPT

    @pl.loop(0, RPT // CHUNK)
    def _(p):
        pltpu.sync_copy(rows3, num_sh.at[pl.ds(base + p * CHUNK, CHUNK)])

    tail0 = RPT - (RPT // CHUNK) * CHUNK
    pltpu.sync_copy(rows3.at[pl.ds(0, tail0)],
                    num_sh.at[pl.ds(base + (RPT // CHUNK) * CHUNK, tail0)])

    for kk in (0, 1, 2):
        slk = pl.ds(kk * CHUNK, CHUNK)
        pltpu.sync_copy(src_hbm.at[wid, slk], src_c[kk])
        pltpu.sync_copy(dst_hbm.at[wid, slk], dst_c[kk])
    for kk in (3, 4, 5):
        slk = pl.ds(kk * CHUNK, CHUNK)
        pltpu.async_copy(src_hbm.at[wid, slk], src_c[kk], isem[kk])
        pltpu.async_copy(dst_hbm.at[wid, slk], dst_c[kk], isem[kk])
    pltpu.async_copy(h_hbm.at[src_c0], rows0, gsem0)
    pltpu.async_copy(h_hbm.at[src_c1], rows1, gsem1)
    pltpu.async_copy(h_hbm.at[src_c2], rows2, gsem2)
    plsc.subcore_barrier()

    mvec = m_v[...]

    def chunk_step(k, b, drain, load4, issue2):
        bd = (b + 3) % 6
        for j in range(CHUNK // 16):
            sl = pl.ds(j * 16, 16)
            s16 = src_c[b][sl]
            d16 = dst_c[b][sl]
            t = (plsc.load_gather(as_v, [s16]) +
                 plsc.load_gather(ad_v, [d16]))
            e = jnp.where(t >= 0.0, t, t * jnp.float32(0.2))
            w = jnp.exp(e - mvec)
            w_v[sl] = w
            dst_s[b][sl] = d16
            plsc.addupdate_scatter(den_v, [d16], w)
        if drain:
            pltpu.make_async_copy(rows[bd], num_sh.at[dst_s[bd]],
                                  ssem[bd]).wait()
        pltpu.make_async_copy(h_hbm.at[src_c[b]], rows[b], gsem[b]).wait()
        if load4:
            sl4 = pl.ds((k + 6) * CHUNK, CHUNK)
            pltpu.async_copy(src_hbm.at[wid, sl4], src_c[b], isem[b])
            pltpu.async_copy(dst_hbm.at[wid, sl4], dst_c[b], isem[b])
        if issue2:
            sl2 = pl.ds((k + 3) * CHUNK, CHUNK)
            pltpu.make_async_copy(src_hbm.at[wid, sl2], src_c[bd],
                                  isem[bd]).wait()
            pltpu.make_async_copy(dst_hbm.at[wid, sl2], dst_c[bd],
                                  isem[bd]).wait()
            pltpu.async_copy(h_hbm.at[src_c[bd]], rows[bd], gsem[bd])
        @pl.loop(0, CHUNK // 16)
        def _(g):
            wvec = w_v[pl.ds(g * 16, 16)]
            for r in range(16):
                wr = wvec.at[jnp.full((16,), r, jnp.int32)].get(
                    mode="promise_in_bounds")
                row = g * 16 + r
                for j in range(8):
                    sl = pl.ds(j * 16, 16)
                    rows[b][row, sl] = rows[b][row, sl] * wr

        pltpu.async_copy(rows[b], num_sh.at[dst_s[b]], ssem[b], add=True)

    chunk_step(0, 0, drain=False, load4=True, issue2=True)
    chunk_step(1, 1, drain=False, load4=True, issue2=True)
    chunk_step(2, 2, drain=False, load4=True, issue2=True)
    chunk_step(3, 3, drain=True, load4=True, issue2=True)
    chunk_step(4, 4, drain=True, load4=True, issue2=True)
    chunk_step(5, 5, drain=True, load4=True, issue2=True)

    @pl.loop(0, (NCHUNK - 12) // 6)
    def _(i):
        k = i * 6 + 6
        chunk_step(k, 0, drain=True, load4=True, issue2=True)
        chunk_step(k + 1, 1, drain=True, load4=True, issue2=True)
        chunk_step(k + 2, 2, drain=True, load4=True, issue2=True)
        chunk_step(k + 3, 3, drain=True, load4=True, issue2=True)
        chunk_step(k + 4, 4, drain=True, load4=True, issue2=True)
        chunk_step(k + 5, 5, drain=True, load4=True, issue2=True)

    chunk_step(NCHUNK - 6, 0, drain=True, load4=False, issue2=True)
    chunk_step(NCHUNK - 5, 1, drain=True, load4=False, issue2=True)
    chunk_step(NCHUNK - 4, 2, drain=True, load4=False, issue2=True)
    chunk_step(NCHUNK - 3, 3, drain=True, load4=False, issue2=False)
    chunk_step(NCHUNK - 2, 4, drain=True, load4=False, issue2=False)
    chunk_step(NCHUNK - 1, 5, drain=True, load4=False, issue2=False)
    pltpu.make_async_copy(rows3, num_sh.at[dst_s3], ssem3).wait()
    pltpu.make_async_copy(rows4, num_sh.at[dst_s4], ssem4).wait()
    pltpu.make_async_copy(rows5, num_sh.at[dst_s5], ssem5).wait()
    plsc.subcore_barrier()

    pltpu.sync_copy(den_v, den_out.at[wid])

    @pl.loop(0, RPT // CHUNK)
    def _(p):
        pltpu.sync_copy(num_sh.at[pl.ds(base + p * CHUNK, CHUNK)],
                        num_out.at[c, pl.ds(base + p * CHUNK, CHUNK)])

    pltpu.sync_copy(num_sh.at[pl.ds(base + (RPT // CHUNK) * CHUNK, tail0)],
                    num_out.at[c, pl.ds(base + (RPT // CHUNK) * CHUNK,
                                        tail0)])


def _sc_edge(src2, dst2, h, at, mvec):
    mesh = plsc.VectorSubcoreMesh(core_axis_name="c", subcore_axis_name="s")
    fn = pl.kernel(
        _sc_body,
        out_type=[
            jax.ShapeDtypeStruct((2, NPAD, 128), jnp.float32),
            jax.ShapeDtypeStruct((NTILES, NPAD), jnp.float32),
        ],
        mesh=mesh,
        compiler_params=pltpu.CompilerParams(needs_layout_passes=False),
        scratch_types=[
            pltpu.VMEM((NPAD,), jnp.float32),
            pltpu.VMEM((NPAD,), jnp.float32),
            pltpu.VMEM((NPAD,), jnp.float32),
            pltpu.VMEM((16,), jnp.float32),
        ] + [pltpu.VMEM((CHUNK,), jnp.int32) for _ in range(18)] + [
            pltpu.VMEM((CHUNK,), jnp.float32),
        ] + [pltpu.VMEM((CHUNK, 128), jnp.float32) for _ in range(6)] + [
            pltpu.VMEM_SHARED((NPAD, 128), jnp.float32),
        ] + [pltpu.SemaphoreType.DMA for _ in range(18)],
    )
    return fn(src2, dst2, h, at, mvec)


def _k2_body(num_ref, den_ref, b_ref, out_ref):
    bvec = b_ref[...]

    @pl.loop(0, NBLK)
    def _(i):
        sl = pl.ds(i * 128, 128)
        ssum = num_ref[0, sl, :] + num_ref[1, sl, :]
        d = jnp.sum(den_ref[:, sl], axis=0, keepdims=True)
        dcol = jnp.transpose(jnp.broadcast_to(d, (128, 128)))
        out_ref[sl, :] = ssum / (dcol + 1e-16) + bvec


def _k2(num, den, b):
    return pl.pallas_call(
        _k2_body,
        out_shape=jax.ShapeDtypeStruct((NPAD, 128), jnp.float32),
        out_specs=pl.BlockSpec((NPAD, 128), lambda: (0, 0)),
    )(num, den, b)


def _pack_aa(a_src, a_dst):
    aa = jnp.zeros((D, 8), jnp.float32)
    return aa.at[:, 0].set(a_src).at[:, 1].set(a_dst)


@jax.jit
def kernel(x, edge_index, W1, a1_src, a1_dst, b1, W2, a2_src, a2_dst, b2):
    ei = edge_index.astype(jnp.int32)
    ei = jnp.pad(ei, ((0, 0), (0, EPAD - E)), constant_values=N)
    src2 = ei[0].reshape(NTILES, TPT)
    dst2 = ei[1].reshape(NTILES, TPT)
    xpad = jnp.pad(x, ((0, NPAD - N), (0, 0)))
    h1, at1, m1 = _k1(xpad, W1, _pack_aa(a1_src, a1_dst))
    num1, den1 = _sc_edge(src2, dst2, h1, at1,
                          jnp.broadcast_to(m1[0, 0], (16,)))
    h2, at2, m2 = _kmid(num1, den1, b1.reshape(1, D), W2,
                        _pack_aa(a2_src, a2_dst))
    num2, den2 = _sc_edge(src2, dst2, h2, at2,
                          jnp.broadcast_to(m2[0, 0], (16,)))
    return _k2(num2, den2, b2.reshape(1, D))[:N]
